# Initial kernel scaffold; baseline (speedup 1.0000x reference)
import jax
import jax.numpy as jnp
from jax.experimental import pallas as pl


def kernel(h, edge_index, edge_type, basis, w_comp, loop_weight, bias, ln_gamma, ln_beta):
    raise NotImplementedError("write your pallas kernel here")


# speedup vs baseline: 20.7644x; 20.7644x over previous
import functools

import jax
import jax.numpy as jnp
from jax import lax
from jax.experimental import pallas as pl
from jax.experimental.pallas import tpu as pltpu
from jax.experimental.pallas import tpu_sc as plsc

N = 10000
E = 320000
IN = 128
OUT = 128
R = 8

NC = 2
NS = 16
NW = NC * NS
EW = E // NW
CH = 128
NFULL = EW // CH
TAIL = EW - NFULL * CH
NP = 10240
RPT = NP // NS

BLK = 1000
NBLK = N // BLK


def _compose_w_body(wc_ref, basis_ref, w_ref):
    for r in range(R):
        acc = wc_ref[r, 0] * basis_ref[0]
        for b in range(1, R):
            acc = acc + wc_ref[r, b] * basis_ref[b]
        w_ref[r] = acc


_compose_w = pl.pallas_call(
    _compose_w_body,
    out_shape=jax.ShapeDtypeStruct((R, IN, OUT), jnp.float32),
    in_specs=[
        pl.BlockSpec(memory_space=pltpu.SMEM),
        pl.BlockSpec((R, IN, OUT), lambda: (0, 0, 0)),
    ],
    out_specs=pl.BlockSpec((R, IN, OUT), lambda: (0, 0, 0)),
)


def _transform_body(h_ref, w_ref, out_ref):
    out_ref[0] = jnp.dot(
        h_ref[...], w_ref[0], preferred_element_type=jnp.float32
    )


_transform = pl.pallas_call(
    _transform_body,
    grid=(NBLK, R),
    out_shape=jax.ShapeDtypeStruct((R, N, OUT), jnp.float32),
    in_specs=[
        pl.BlockSpec((BLK, IN), lambda nb, r: (nb, 0)),
        pl.BlockSpec((1, IN, OUT), lambda nb, r: (r, 0, 0)),
    ],
    out_specs=pl.BlockSpec((1, BLK, OUT), lambda nb, r: (r, nb, 0)),
)


_mesh = plsc.VectorSubcoreMesh(
    core_axis_name="c", subcore_axis_name="s", num_cores=NC, num_subcores=NS
)


@functools.partial(
    pl.kernel,
    mesh=_mesh,
    out_type=jax.ShapeDtypeStruct((NC, NP, OUT), jnp.float32),
    scratch_types=[
        pltpu.VMEM((EW,), jnp.int32),
        pltpu.VMEM((EW,), jnp.int32),
        pltpu.VMEM((EW,), jnp.int32),
        pltpu.VMEM((CH,), jnp.int32),
        pltpu.VMEM((CH, OUT), jnp.float32),
        pltpu.VMEM((TAIL,), jnp.int32),
        pltpu.VMEM((TAIL, OUT), jnp.float32),
        pltpu.VMEM_SHARED((NP, OUT), jnp.float32),
        pltpu.SemaphoreType.DMA,
    ],
)
def _sc_agg(allt_hbm, src_hbm, dst_hbm, typ_hbm, zeros_hbm, out_hbm,
            gidx_v, dst_v, typ_v, ddst_v, rows_v, tidx_v, trows_v,
            agg_sh, sem):
    c = lax.axis_index("c")
    s = lax.axis_index("s")
    wid = c * NS + s
    eoff = wid * EW
    pltpu.sync_copy(src_hbm.at[pl.ds(eoff, EW)], gidx_v)
    pltpu.sync_copy(dst_hbm.at[pl.ds(eoff, EW)], dst_v)
    pltpu.sync_copy(typ_hbm.at[pl.ds(eoff, EW)], typ_v)
    roff = s * RPT
    pltpu.sync_copy(zeros_hbm.at[pl.ds(roff, RPT)],
                    agg_sh.at[pl.ds(roff, RPT)])

    def _mk_gidx(i, carry):
        sl = pl.ds(i * 16, 16)
        gidx_v[sl] = typ_v[sl] * N + gidx_v[sl]
        return carry

    lax.fori_loop(0, EW // 16, _mk_gidx, 0)

    plsc.subcore_barrier()

    def _chunk(k, carry):
        base = k * CH
        cp = pltpu.async_copy(
            allt_hbm.at[gidx_v.at[pl.ds(base, CH)]], rows_v, sem
        )
        for j in range(CH // 16):
            ddst_v[pl.ds(j * 16, 16)] = dst_v[pl.ds(base + j * 16, 16)]
        cp.wait()
        pltpu.sync_copy(rows_v, agg_sh.at[ddst_v], add=True)
        return carry

    lax.fori_loop(0, NFULL, _chunk, 0)

    tbase = NFULL * CH
    cpt = pltpu.async_copy(
        allt_hbm.at[gidx_v.at[pl.ds(tbase, TAIL)]], trows_v, sem
    )
    for j in range(TAIL // 16):
        tidx_v[pl.ds(j * 16, 16)] = dst_v[pl.ds(tbase + j * 16, 16)]
    cpt.wait()
    pltpu.sync_copy(trows_v, agg_sh.at[tidx_v], add=True)

    plsc.subcore_barrier()
    pltpu.sync_copy(agg_sh.at[pl.ds(roff, RPT)],
                    out_hbm.at[c, pl.ds(roff, RPT)])


def _final_body(p_ref, h_ref, lw_ref, b_ref, g_ref, be_ref, o_ref):
    x = (
        p_ref[0]
        + p_ref[1]
        + jnp.dot(h_ref[...], lw_ref[...], preferred_element_type=jnp.float32)
        + b_ref[...]
    )
    mu = jnp.mean(x, axis=1, keepdims=True)
    xc = x - mu
    var = jnp.mean(xc * xc, axis=1, keepdims=True)
    y = xc * lax.rsqrt(var + 1e-5) * g_ref[...] + be_ref[...]
    o_ref[...] = jnp.maximum(y, 0.0)


_final = pl.pallas_call(
    _final_body,
    grid=(NBLK,),
    out_shape=jax.ShapeDtypeStruct((N, OUT), jnp.float32),
    in_specs=[
        pl.BlockSpec((NC, BLK, OUT), lambda nb: (0, nb, 0)),
        pl.BlockSpec((BLK, IN), lambda nb: (nb, 0)),
        pl.BlockSpec((IN, OUT), lambda nb: (0, 0)),
        pl.BlockSpec((1, OUT), lambda nb: (0, 0)),
        pl.BlockSpec((1, OUT), lambda nb: (0, 0)),
        pl.BlockSpec((1, OUT), lambda nb: (0, 0)),
    ],
    out_specs=pl.BlockSpec((BLK, OUT), lambda nb: (nb, 0)),
)


def kernel(h, edge_index, edge_type, basis, w_comp, loop_weight, bias,
           ln_gamma, ln_beta):
    W = _compose_w(w_comp, basis)
    allt = _transform(h, W)
    allt2d = allt.reshape(R * N, OUT)
    src = edge_index[0]
    dst = edge_index[1]
    zeros = jnp.zeros((NP, OUT), jnp.float32)
    parts = _sc_agg(allt2d, src, dst, edge_type, zeros)
    return _final(
        parts,
        h,
        loop_weight,
        bias.reshape(1, OUT),
        ln_gamma.reshape(1, OUT),
        ln_beta.reshape(1, OUT),
    )


# speedup vs baseline: 28.4674x; 1.3710x over previous
import functools

import jax
import jax.numpy as jnp
from jax import lax
from jax.experimental import pallas as pl
from jax.experimental.pallas import tpu as pltpu
from jax.experimental.pallas import tpu_sc as plsc

N = 10000
E = 320000
IN = 128
OUT = 128
R = 8

NC = 2
NS = 16
NW = NC * NS
EW = E // NW
CH = 128
NFULL = EW // CH
TAIL = EW - NFULL * CH
QW = 2000
NP = 10240
RPT = NP // NS

BLK = 1000
NBLK = N // BLK


def _transform_body(wc_ref, basis_ref, h_ref, out_ref):
    r = pl.program_id(1)
    w = wc_ref[r, 0] * basis_ref[0]
    for b in range(1, R):
        w = w + wc_ref[r, b] * basis_ref[b]
    out_ref[0] = jnp.dot(h_ref[...], w, preferred_element_type=jnp.float32)


_transform = pl.pallas_call(
    _transform_body,
    grid=(NBLK, R),
    out_shape=jax.ShapeDtypeStruct((R, N, OUT), jnp.float32),
    in_specs=[
        pl.BlockSpec(memory_space=pltpu.SMEM),
        pl.BlockSpec((R, IN, OUT), lambda nb, r: (0, 0, 0)),
        pl.BlockSpec((BLK, IN), lambda nb, r: (nb, 0)),
    ],
    out_specs=pl.BlockSpec((1, BLK, OUT), lambda nb, r: (r, nb, 0)),
)


_mesh = plsc.VectorSubcoreMesh(
    core_axis_name="c", subcore_axis_name="s", num_cores=NC, num_subcores=NS
)


@functools.partial(
    pl.kernel,
    mesh=_mesh,
    out_type=jax.ShapeDtypeStruct((NC, NP, OUT), jnp.float32),
    scratch_types=[
        pltpu.VMEM_SHARED((NP, OUT), jnp.float32),
        pltpu.VMEM((EW,), jnp.int32),
        pltpu.VMEM((QW,), jnp.int32),
        pltpu.VMEM((CH,), jnp.int32),
        pltpu.VMEM((CH,), jnp.int32),
        pltpu.VMEM((CH, OUT), jnp.float32),
        pltpu.VMEM((CH, OUT), jnp.float32),
        pltpu.VMEM((TAIL,), jnp.int32),
        pltpu.VMEM((TAIL, OUT), jnp.float32),
        pltpu.SemaphoreType.DMA,
        pltpu.SemaphoreType.DMA,
    ],
)
def _sc_agg(allt_hbm, src_hbm, dst_hbm, typ_hbm, out_hbm,
            agg_sh, gidx_v, qbuf_v, ddst_a, ddst_b, rows_a, rows_b,
            tidx_v, trows_v, sem_a, sem_b):
    c = lax.axis_index("c")
    s = lax.axis_index("s")
    wid = c * NS + s
    eoff = wid * EW
    pltpu.sync_copy(src_hbm.at[pl.ds(eoff, EW)], gidx_v)

    zv = jnp.zeros((16,), jnp.float32)

    def _zero(i, carry):
        rows_a[i // (OUT // 16), pl.ds((i % (OUT // 16)) * 16, 16)] = zv
        return carry

    lax.fori_loop(0, CH * OUT // 16, _zero, 0)
    roff = s * RPT
    for i in range(RPT // CH):
        pltpu.sync_copy(rows_a, agg_sh.at[pl.ds(roff + i * CH, CH)])

    for p in range(EW // QW):
        pltpu.sync_copy(typ_hbm.at[pl.ds(eoff + p * QW, QW)], qbuf_v)

        def _mk_gidx(i, carry):
            so = pl.ds(p * QW + i * 16, 16)
            gidx_v[so] = qbuf_v[pl.ds(i * 16, 16)] * N + gidx_v[so]
            return carry

        lax.fori_loop(0, QW // 16, _mk_gidx, 0)

    plsc.subcore_barrier()

    def _fire(k, rows_ref, ddst_ref, sem):
        pltpu.async_copy(
            allt_hbm.at[gidx_v.at[pl.ds(k * CH, CH)]], rows_ref, sem
        )
        pltpu.async_copy(dst_hbm.at[pl.ds(eoff + k * CH, CH)], ddst_ref, sem)

    def _wait(k, rows_ref, ddst_ref, sem):
        pltpu.make_async_copy(
            allt_hbm.at[gidx_v.at[pl.ds(k * CH, CH)]], rows_ref, sem
        ).wait()
        pltpu.make_async_copy(
            dst_hbm.at[pl.ds(eoff + k * CH, CH)], ddst_ref, sem
        ).wait()

    _fire(0, rows_a, ddst_a, sem_a)

    def _pair(t, carry):
        ka = 2 * t
        kb = 2 * t + 1
        _fire(kb, rows_b, ddst_b, sem_b)
        _wait(ka, rows_a, ddst_a, sem_a)
        pltpu.sync_copy(rows_a, agg_sh.at[ddst_a], add=True)

        @pl.when(t < NFULL // 2 - 1)
        def _():
            _fire(ka + 2, rows_a, ddst_a, sem_a)

        _wait(kb, rows_b, ddst_b, sem_b)
        pltpu.sync_copy(rows_b, agg_sh.at[ddst_b], add=True)
        return carry

    lax.fori_loop(0, NFULL // 2, _pair, 0)

    tbase = NFULL * CH
    cpt = pltpu.async_copy(
        allt_hbm.at[gidx_v.at[pl.ds(tbase, TAIL)]], trows_v, sem_a
    )
    pltpu.sync_copy(dst_hbm.at[pl.ds(eoff + tbase, TAIL)], tidx_v)
    cpt.wait()
    pltpu.sync_copy(trows_v, agg_sh.at[tidx_v], add=True)

    plsc.subcore_barrier()
    pltpu.sync_copy(agg_sh.at[pl.ds(roff, RPT)],
                    out_hbm.at[c, pl.ds(roff, RPT)])


def _final_body(p_ref, h_ref, lw_ref, b_ref, g_ref, be_ref, o_ref):
    x = (
        p_ref[0]
        + p_ref[1]
        + jnp.dot(h_ref[...], lw_ref[...], preferred_element_type=jnp.float32)
        + b_ref[...]
    )
    mu = jnp.mean(x, axis=1, keepdims=True)
    xc = x - mu
    var = jnp.mean(xc * xc, axis=1, keepdims=True)
    y = xc * lax.rsqrt(var + 1e-5) * g_ref[...] + be_ref[...]
    o_ref[...] = jnp.maximum(y, 0.0)


_final = pl.pallas_call(
    _final_body,
    grid=(NBLK,),
    out_shape=jax.ShapeDtypeStruct((N, OUT), jnp.float32),
    in_specs=[
        pl.BlockSpec((NC, BLK, OUT), lambda nb: (0, nb, 0)),
        pl.BlockSpec((BLK, IN), lambda nb: (nb, 0)),
        pl.BlockSpec((IN, OUT), lambda nb: (0, 0)),
        pl.BlockSpec((1, OUT), lambda nb: (0, 0)),
        pl.BlockSpec((1, OUT), lambda nb: (0, 0)),
        pl.BlockSpec((1, OUT), lambda nb: (0, 0)),
    ],
    out_specs=pl.BlockSpec((BLK, OUT), lambda nb: (nb, 0)),
)


def kernel(h, edge_index, edge_type, basis, w_comp, loop_weight, bias,
           ln_gamma, ln_beta):
    allt = _transform(w_comp, basis, h)
    allt2d = allt.reshape(R * N, OUT)
    src = edge_index[0]
    dst = edge_index[1]
    parts = _sc_agg(allt2d, src, dst, edge_type)
    return _final(
        parts,
        h,
        loop_weight,
        bias.reshape(1, OUT),
        ln_gamma.reshape(1, OUT),
        ln_beta.reshape(1, OUT),
    )


# speedup vs baseline: 30.4656x; 1.0702x over previous
import functools

import jax
import jax.numpy as jnp
from jax import lax
from jax.experimental import pallas as pl
from jax.experimental.pallas import tpu as pltpu
from jax.experimental.pallas import tpu_sc as plsc

N = 10000
E = 320000
IN = 128
OUT = 128
R = 8

NC = 2
NS = 16
NW = NC * NS
EW = E // NW
CH = 128
NFULL = EW // CH
TAIL = EW - NFULL * CH
QW = 2000
NP = 10240
RPT = NP // NS

BLK = 1000
NBLK = N // BLK


def _transform_body(wc_ref, basis_ref, h_ref, out_ref):
    r = pl.program_id(1)
    w = wc_ref[r, 0] * basis_ref[0]
    for b in range(1, R):
        w = w + wc_ref[r, b] * basis_ref[b]
    out_ref[0] = jnp.dot(
        h_ref[...].astype(jnp.bfloat16),
        w.astype(jnp.bfloat16),
        preferred_element_type=jnp.float32,
    )


_transform = pl.pallas_call(
    _transform_body,
    grid=(NBLK, R),
    out_shape=jax.ShapeDtypeStruct((R, N, OUT), jnp.float32),
    in_specs=[
        pl.BlockSpec(memory_space=pltpu.SMEM),
        pl.BlockSpec((R, IN, OUT), lambda nb, r: (0, 0, 0)),
        pl.BlockSpec((BLK, IN), lambda nb, r: (nb, 0)),
    ],
    out_specs=pl.BlockSpec((1, BLK, OUT), lambda nb, r: (r, nb, 0)),
)


_mesh = plsc.VectorSubcoreMesh(
    core_axis_name="c", subcore_axis_name="s", num_cores=NC, num_subcores=NS
)


@functools.partial(
    pl.kernel,
    mesh=_mesh,
    out_type=jax.ShapeDtypeStruct((NC, NP, OUT), jnp.float32),
    scratch_types=[
        pltpu.VMEM_SHARED((NP, OUT), jnp.float32),
        pltpu.VMEM((EW,), jnp.int32),
        pltpu.VMEM((QW,), jnp.int32),
        pltpu.VMEM((CH,), jnp.int32),
        pltpu.VMEM((CH,), jnp.int32),
        pltpu.VMEM((CH, OUT), jnp.float32),
        pltpu.VMEM((CH, OUT), jnp.float32),
        pltpu.VMEM((TAIL,), jnp.int32),
        pltpu.VMEM((TAIL, OUT), jnp.float32),
        pltpu.SemaphoreType.DMA,
        pltpu.SemaphoreType.DMA,
    ],
)
def _sc_agg(allt_hbm, edge_hbm, typ_hbm, out_hbm,
            agg_sh, gidx_v, qbuf_v, ddst_a, ddst_b, rows_a, rows_b,
            tidx_v, trows_v, sem_a, sem_b):
    c = lax.axis_index("c")
    s = lax.axis_index("s")
    wid = c * NS + s
    eoff = wid * EW
    pltpu.sync_copy(edge_hbm.at[pl.ds(eoff, EW)], gidx_v)

    zv = jnp.zeros((16,), jnp.float32)

    def _zero(i, carry):
        for j in range(OUT // 16):
            rows_a[i, pl.ds(j * 16, 16)] = zv
        return carry

    lax.fori_loop(0, CH, _zero, 0)
    roff = s * RPT
    for i in range(RPT // CH):
        pltpu.sync_copy(rows_a, agg_sh.at[pl.ds(roff + i * CH, CH)])

    for p in range(EW // QW):
        pltpu.sync_copy(typ_hbm.at[pl.ds(eoff + p * QW, QW)], qbuf_v)

        def _mk_gidx(i, carry):
            so = pl.ds(p * QW + i * 16, 16)
            gidx_v[so] = qbuf_v[pl.ds(i * 16, 16)] * N + gidx_v[so]
            return carry

        lax.fori_loop(0, QW // 16, _mk_gidx, 0)

    plsc.subcore_barrier()

    def _fire(k, rows_ref, ddst_ref, sem):
        pltpu.async_copy(
            allt_hbm.at[gidx_v.at[pl.ds(k * CH, CH)]], rows_ref, sem
        )
        pltpu.async_copy(edge_hbm.at[pl.ds(E + eoff + k * CH, CH)], ddst_ref, sem)

    def _wait(k, rows_ref, ddst_ref, sem):
        pltpu.make_async_copy(
            allt_hbm.at[gidx_v.at[pl.ds(k * CH, CH)]], rows_ref, sem
        ).wait()
        pltpu.make_async_copy(
            edge_hbm.at[pl.ds(E + eoff + k * CH, CH)], ddst_ref, sem
        ).wait()

    _fire(0, rows_a, ddst_a, sem_a)

    def _pair(t, carry):
        ka = 2 * t
        kb = 2 * t + 1
        _fire(kb, rows_b, ddst_b, sem_b)
        _wait(ka, rows_a, ddst_a, sem_a)
        pltpu.sync_copy(rows_a, agg_sh.at[ddst_a], add=True)

        @pl.when(t < NFULL // 2 - 1)
        def _():
            _fire(ka + 2, rows_a, ddst_a, sem_a)

        _wait(kb, rows_b, ddst_b, sem_b)
        pltpu.sync_copy(rows_b, agg_sh.at[ddst_b], add=True)
        return carry

    lax.fori_loop(0, NFULL // 2, _pair, 0)

    tbase = NFULL * CH
    cpt = pltpu.async_copy(
        allt_hbm.at[gidx_v.at[pl.ds(tbase, TAIL)]], trows_v, sem_a
    )
    pltpu.sync_copy(edge_hbm.at[pl.ds(E + eoff + tbase, TAIL)], tidx_v)
    cpt.wait()
    pltpu.sync_copy(trows_v, agg_sh.at[tidx_v], add=True)

    plsc.subcore_barrier()
    pltpu.sync_copy(agg_sh.at[pl.ds(roff, RPT)],
                    out_hbm.at[c, pl.ds(roff, RPT)])


def _final_body(p_ref, h_ref, lw_ref, b_ref, g_ref, be_ref, o_ref):
    x = (
        p_ref[0]
        + p_ref[1]
        + jnp.dot(h_ref[...], lw_ref[...], preferred_element_type=jnp.float32)
        + b_ref[...]
    )
    mu = jnp.mean(x, axis=1, keepdims=True)
    xc = x - mu
    var = jnp.mean(xc * xc, axis=1, keepdims=True)
    y = xc * lax.rsqrt(var + 1e-5) * g_ref[...] + be_ref[...]
    o_ref[...] = jnp.maximum(y, 0.0)


_final = pl.pallas_call(
    _final_body,
    grid=(NBLK,),
    out_shape=jax.ShapeDtypeStruct((N, OUT), jnp.float32),
    in_specs=[
        pl.BlockSpec((NC, BLK, OUT), lambda nb: (0, nb, 0)),
        pl.BlockSpec((BLK, IN), lambda nb: (nb, 0)),
        pl.BlockSpec((IN, OUT), lambda nb: (0, 0)),
        pl.BlockSpec((1, OUT), lambda nb: (0, 0)),
        pl.BlockSpec((1, OUT), lambda nb: (0, 0)),
        pl.BlockSpec((1, OUT), lambda nb: (0, 0)),
    ],
    out_specs=pl.BlockSpec((BLK, OUT), lambda nb: (nb, 0)),
)


def kernel(h, edge_index, edge_type, basis, w_comp, loop_weight, bias,
           ln_gamma, ln_beta):
    allt = _transform(w_comp, basis, h)
    allt2d = allt.reshape(R * N, OUT)
    parts = _sc_agg(allt2d, edge_index.reshape(2 * E), edge_type)
    return _final(
        parts,
        h,
        loop_weight,
        bias.reshape(1, OUT),
        ln_gamma.reshape(1, OUT),
        ln_beta.reshape(1, OUT),
    )
